# Initial kernel scaffold; baseline (speedup 1.0000x reference)
import functools

import jax
import jax.numpy as jnp
from jax import lax
from jax.experimental import pallas as pl
from jax.experimental.pallas import tpu as pltpu
from jax.experimental.pallas import tpu_sc as plsc

N = 100000
NC, NS, LANES = 2, 16, 16
NW = NC * NS
NPAD = 100352
PER_TILE = NPAD // NW
HROWS = 168
HSIZE = HROWS * 128
FP_OFF = 10240
JUNK = 20480


def _sc_partial_hists(preds_pad, targets_pad):
    mesh = plsc.VectorSubcoreMesh(
        core_axis_name="c", subcore_axis_name="s",
        num_cores=NC, num_subcores=NS)

    @functools.partial(
        pl.kernel,
        out_type=jax.ShapeDtypeStruct((NW, HSIZE), jnp.float32),
        mesh=mesh,
        scratch_types=[
            pltpu.VMEM((PER_TILE,), jnp.float32),
            pltpu.VMEM((PER_TILE,), jnp.float32),
            pltpu.VMEM((HSIZE,), jnp.float32),
            pltpu.SemaphoreType.DMA,
        ],
    )
    def hist_kernel(p_hbm, t_hbm, o_hbm, p_v, t_v, hist_v, sem):
        wid = lax.axis_index("s") * NC + lax.axis_index("c")
        base = wid * PER_TILE
        p_cp = pltpu.async_copy(p_hbm.at[pl.ds(base, PER_TILE)], p_v, sem)
        t_cp = pltpu.async_copy(t_hbm.at[pl.ds(base, PER_TILE)], t_v, sem)

        zeros16 = jnp.zeros((LANES,), jnp.float32)

        @pl.loop(0, HSIZE, step=LANES)
        def _(i):
            hist_v[pl.ds(i, LANES)] = zeros16

        p_cp.wait()
        t_cp.wait()

        lane = lax.iota(jnp.int32, LANES)

        @pl.loop(0, PER_TILE, step=LANES)
        def _(i):
            x = p_v[pl.ds(i, LANES)]
            t = t_v[pl.ds(i, LANES)]
            e = jnp.exp(-jnp.abs(x))
            sig = jnp.where(x >= 0.0, 1.0 / (1.0 + e), e / (1.0 + e))
            b = (10000.0 * sig).astype(jnp.int32)
            idx = b + jnp.where(t >= 0.5, 0, FP_OFF)
            g = base + i + lane
            idx = jnp.where(g < N, idx, JUNK + lane)
            cnt, last = plsc.scan_count(idx)
            plsc.addupdate_scatter(hist_v, [idx], cnt.astype(jnp.float32),
                                   mask=last)

        pltpu.async_copy(hist_v, o_hbm.at[wid], sem).wait()

    return hist_kernel(preds_pad, targets_pad)


def _tc_auc(partials3):
    def body(pref, oref):
        h = jnp.sum(pref[...], axis=0)
        tp = h[0:80, :]
        fp = h[80:160, :]
        ii = lax.broadcasted_iota(jnp.int32, (128, 128), 0)
        jj = lax.broadcasted_iota(jnp.int32, (128, 128), 1)
        m_low = (ii >= jj).astype(jnp.float32)
        u_strict = (jj > ii).astype(jnp.float32)
        tp128 = jnp.concatenate(
            [tp, jnp.zeros((48, 128), jnp.float32)], axis=0)
        s_in = jax.lax.dot(tp, m_low, preferred_element_type=jnp.float32)
        a = jax.lax.dot(u_strict, tp128,
                        preferred_element_type=jnp.float32)[0:80, :]
        s_incl = s_in + jnp.sum(a, axis=1, keepdims=True)
        cross = jnp.sum(fp * s_incl)
        ties = jnp.sum(fp * tp)
        tp_total = jnp.sum(tp)
        fp_total = jnp.sum(fp)
        oref[0, 0] = (cross - 0.5 * ties) / (tp_total * fp_total)

    out = pl.pallas_call(
        body,
        out_shape=jax.ShapeDtypeStruct((1, 1), jnp.float32),
    )(partials3)
    return out[0, 0]


@jax.jit
def kernel(preds, targets):
    preds_pad = jnp.pad(preds.reshape(-1), (0, NPAD - N))
    targets_pad = jnp.pad(targets.reshape(-1), (0, NPAD - N))
    partials = _sc_partial_hists(preds_pad, targets_pad)
    return _tc_auc(partials.reshape(NW, HROWS, 128))


# speedup vs baseline: 6.3971x; 6.3971x over previous
import functools

import jax
import jax.numpy as jnp
from jax import lax
from jax.experimental import pallas as pl
from jax.experimental.pallas import tpu as pltpu
from jax.experimental.pallas import tpu_sc as plsc

N = 100000
NC, NS, LANES = 2, 16, 16
NW = NC * NS
NPAD = 100352
PER_TILE = NPAD // NW
HROWS = 168
HSIZE = HROWS * 128
FP_OFF = 10240
JUNK = 20480


def _sc_partial_hists(preds_pad, targets_pad):
    mesh = plsc.VectorSubcoreMesh(
        core_axis_name="c", subcore_axis_name="s",
        num_cores=NC, num_subcores=NS)

    @functools.partial(
        pl.kernel,
        out_type=jax.ShapeDtypeStruct((NW, HSIZE), jnp.float32),
        mesh=mesh,
        scratch_types=[
            pltpu.VMEM((PER_TILE,), jnp.float32),
            pltpu.VMEM((PER_TILE,), jnp.float32),
            pltpu.VMEM((HSIZE,), jnp.float32),
            pltpu.SemaphoreType.DMA,
        ],
        compiler_params=pltpu.CompilerParams(needs_layout_passes=False),
    )
    def hist_kernel(p_hbm, t_hbm, o_hbm, p_v, t_v, hist_v, sem):
        wid = lax.axis_index("s") * NC + lax.axis_index("c")
        base = wid * PER_TILE
        p_cp = pltpu.async_copy(p_hbm.at[pl.ds(base, PER_TILE)], p_v, sem)
        t_cp = pltpu.async_copy(t_hbm.at[pl.ds(base, PER_TILE)], t_v, sem)

        zeros16 = jnp.zeros((LANES,), jnp.float32)

        @pl.loop(0, HSIZE, step=LANES)
        def _(i):
            hist_v[pl.ds(i, LANES)] = zeros16

        p_cp.wait()
        t_cp.wait()

        lane = lax.iota(jnp.int32, LANES)

        @pl.loop(0, PER_TILE, step=LANES)
        def _(i):
            x = p_v[pl.ds(i, LANES)]
            t = t_v[pl.ds(i, LANES)]
            e = jnp.exp(-jnp.abs(x))
            sig = jnp.where(x >= 0.0, 1.0 / (1.0 + e), e / (1.0 + e))
            b = (10000.0 * sig).astype(jnp.int32)
            idx = b + jnp.where(t >= 0.5, 0, FP_OFF)
            g = base + i + lane
            idx = jnp.where(g < N, idx, JUNK + lane)
            cnt, last = plsc.scan_count(idx)
            plsc.addupdate_scatter(hist_v, [idx], cnt.astype(jnp.float32),
                                   mask=last)

        pltpu.async_copy(hist_v, o_hbm.at[wid], sem).wait()

    return hist_kernel(preds_pad, targets_pad)


def _tc_auc(partials3):
    def body(pref, oref):
        h = jnp.sum(pref[...], axis=0)
        tp = h[0:80, :]
        fp = h[80:160, :]
        ii = lax.broadcasted_iota(jnp.int32, (128, 128), 0)
        jj = lax.broadcasted_iota(jnp.int32, (128, 128), 1)
        m_low = (ii >= jj).astype(jnp.float32)
        u_strict = (jj > ii).astype(jnp.float32)
        tp128 = jnp.concatenate(
            [tp, jnp.zeros((48, 128), jnp.float32)], axis=0)
        s_in = jax.lax.dot(tp, m_low, preferred_element_type=jnp.float32)
        a = jax.lax.dot(u_strict, tp128,
                        preferred_element_type=jnp.float32)[0:80, :]
        s_incl = s_in + jnp.sum(a, axis=1, keepdims=True)
        cross = jnp.sum(fp * s_incl)
        ties = jnp.sum(fp * tp)
        tp_total = jnp.sum(tp)
        fp_total = jnp.sum(fp)
        auc = (cross - 0.5 * ties) / (tp_total * fp_total)
        oref[...] = jnp.broadcast_to(auc, (1, 1))

    out = pl.pallas_call(
        body,
        out_shape=jax.ShapeDtypeStruct((1, 1), jnp.float32),
    )(partials3)
    return out[0, 0]


@jax.jit
def kernel(preds, targets):
    preds_pad = jnp.pad(preds.reshape(-1), (0, NPAD - N))
    targets_pad = jnp.pad(targets.reshape(-1), (0, NPAD - N))
    partials = _sc_partial_hists(preds_pad, targets_pad)
    return _tc_auc(partials.reshape(NW, HROWS, 128))
